# baseline (speedup 1.0000x reference)
import jax
import jax.numpy as jnp
from jax import lax
from jax.experimental import pallas as pl
from jax.experimental.pallas import tpu as pltpu
from jax.experimental.pallas import tpu_sc as plsc

N = 100000
E = 400000
HID = 64
NPG = 5
ROWS_BLK = 2000
HEAD_BLK = 2000

EP = 400384
EPT = EP // 32
EPTB = EPT + 32
NG = EPT // 16
NCHUNK = 8
NP = 102400
CHUNK = NP // NCHUNK
STRIPE = CHUNK // 16
ZR = 160
EB = 256
NB = EPTB // EB


def _wid(c, s):
    return c * 16 + s


def _sc_norm_body(dis_hbm, src_hbm, dst_hbm, out_hbm,
                  src_v, dst_v, nrm_v, bs_v, bd_v, ga, gb, sem, sem2):
    c = lax.axis_index("c")
    s = lax.axis_index("s")
    wid = _wid(c, s)
    base = wid * EPT
    pltpu.sync_copy(src_hbm.at[pl.ds(base, EPT)], src_v.at[pl.ds(0, EPT)])
    pltpu.sync_copy(dst_hbm.at[pl.ds(base, EPT)], dst_v.at[pl.ds(0, EPT)])

    z16i = jnp.zeros((16,), jnp.int32)
    for g in range(EPT // 16, EPTB // 16):
        src_v[pl.ds(g * 16, 16)] = z16i
        dst_v[pl.ds(g * 16, 16)] = z16i

    def batch(bi, carry):
        for j in range(EB // 16):
            bs_v[pl.ds(j * 16, 16)] = src_v[pl.ds(bi * EB + j * 16, 16)]
            bd_v[pl.ds(j * 16, 16)] = dst_v[pl.ds(bi * EB + j * 16, 16)]
        cp1 = pltpu.async_copy(dis_hbm.at[bs_v], ga, sem)
        cp2 = pltpu.async_copy(dis_hbm.at[bd_v], gb, sem2)
        cp1.wait()
        cp2.wait()
        for j in range(EB // 16):
            p = (base + bi * EB + j * 16
                 + lax.broadcasted_iota(jnp.int32, (16,), 0))
            v = ga[pl.ds(j * 16, 16)] * gb[pl.ds(j * 16, 16)]
            nrm_v[pl.ds(bi * EB + j * 16, 16)] = jnp.where(p < E, v, 0.0)
        return carry
    lax.fori_loop(0, EPTB // EB, batch, 0)

    pltpu.sync_copy(nrm_v.at[pl.ds(0, EPT)], out_hbm.at[pl.ds(base, EPT)])


def _sc_norm(dis, srcp, dstp):
    return pl.kernel(
        _sc_norm_body,
        out_type=jax.ShapeDtypeStruct((EP,), jnp.float32),
        mesh=plsc.VectorSubcoreMesh(core_axis_name="c", subcore_axis_name="s"),
        compiler_params=pltpu.CompilerParams(use_tc_tiling_on_sc=False),
        scratch_types=[
            pltpu.VMEM((EPTB,), jnp.int32),
            pltpu.VMEM((EPTB,), jnp.int32),
            pltpu.VMEM((EPTB,), jnp.float32),
            pltpu.VMEM((EB,), jnp.int32),
            pltpu.VMEM((EB,), jnp.int32),
            pltpu.VMEM((EB,), jnp.float32),
            pltpu.VMEM((EB,), jnp.float32),
            pltpu.SemaphoreType.DMA,
            pltpu.SemaphoreType.DMA,
        ],
    )(dis, srcp, dstp)


def _sc_agg_body(m_hbm, src_hbm, dst_hbm, norm_hbm, out_hbm,
                 src_v, dst_v, norm_v, bsrc, bdst, bnrm, rows, zrow, acc, sem):
    c = lax.axis_index("c")
    s = lax.axis_index("s")
    base_e = _wid(c, s) * EPT
    pltpu.sync_copy(src_hbm.at[pl.ds(base_e, EPT)], src_v.at[pl.ds(0, EPT)])
    pltpu.sync_copy(dst_hbm.at[pl.ds(base_e, EPT)], dst_v.at[pl.ds(0, EPT)])
    pltpu.sync_copy(norm_hbm.at[pl.ds(base_e, EPT)], norm_v.at[pl.ds(0, EPT)])

    z16i = jnp.zeros((16,), jnp.int32)
    z16f = jnp.zeros((16,), jnp.float32)
    for g in range(EPT // 16, EPTB // 16):
        src_v[pl.ds(g * 16, 16)] = z16i
        dst_v[pl.ds(g * 16, 16)] = z16i
        norm_v[pl.ds(g * 16, 16)] = z16f

    def init_zrow(r, carry):
        for j in range(HID // 16):
            zrow[r, pl.ds(j * 16, 16)] = z16f
        return carry
    lax.fori_loop(0, ZR, init_zrow, 0)

    def zero_stripe():
        for k in range(STRIPE // ZR):
            pltpu.sync_copy(zrow, acc.at[pl.ds(s * STRIPE + k * ZR, ZR), :])

    zero_stripe()
    plsc.subcore_barrier()

    for cidx in range(NCHUNK):
        lo = cidx * CHUNK

        def batch(bi, carry):
            for j in range(EB // 16):
                d16 = dst_v[pl.ds(bi * EB + j * 16, 16)]
                n16 = norm_v[pl.ds(bi * EB + j * 16, 16)]
                msk = (d16 >= lo) & (d16 < lo + CHUNK)
                bsrc[pl.ds(j * 16, 16)] = src_v[pl.ds(bi * EB + j * 16, 16)]
                bdst[pl.ds(j * 16, 16)] = jnp.clip(d16 - lo, 0, CHUNK - 1)
                bnrm[pl.ds(j * 16, 16)] = jnp.where(msk, n16, 0.0)
            pltpu.async_copy(m_hbm.at[bsrc], rows, sem).wait()

            def scale(gg, carry2):
                nv = bnrm[pl.ds(gg * 16, 16)]
                for k in range(16):
                    r = gg * 16 + k
                    nrm = nv[k]
                    for j in range(HID // 16):
                        rows[r, pl.ds(j * 16, 16)] = (
                            rows[r, pl.ds(j * 16, 16)] * nrm
                        )
                return carry2
            lax.fori_loop(0, EB // 16, scale, 0)
            pltpu.sync_copy(rows, acc.at[bdst], add=True)
            return carry
        lax.fori_loop(0, NB, batch, 0)

        plsc.subcore_barrier()
        for k in range(STRIPE // ZR):
            pltpu.sync_copy(acc.at[pl.ds(s * STRIPE + k * ZR, ZR), :],
                            out_hbm.at[c, pl.ds(lo + s * STRIPE + k * ZR, ZR), :])
        zero_stripe()
        plsc.subcore_barrier()


def _sc_aggregate(m, srcp, dstp, normp):
    return pl.kernel(
        _sc_agg_body,
        out_type=jax.ShapeDtypeStruct((2, NP, HID), jnp.float32),
        mesh=plsc.VectorSubcoreMesh(core_axis_name="c", subcore_axis_name="s"),
        compiler_params=pltpu.CompilerParams(use_tc_tiling_on_sc=False),
        scratch_types=[
            pltpu.VMEM((EPTB,), jnp.int32),
            pltpu.VMEM((EPTB,), jnp.int32),
            pltpu.VMEM((EPTB,), jnp.float32),
            pltpu.VMEM((EB,), jnp.int32),
            pltpu.VMEM((EB,), jnp.int32),
            pltpu.VMEM((EB,), jnp.float32),
            pltpu.VMEM((EB, HID), jnp.float32),
            pltpu.VMEM((ZR, HID), jnp.float32),
            pltpu.VMEM_SHARED((CHUNK, HID), jnp.float32),
            pltpu.SemaphoreType.DMA,
        ],
    )(m, srcp, dstp, normp)


def _gelu(t):
    return 0.5 * t * (1.0 + jax.lax.erf(t * 0.7071067811865475))


def _in_proj_body(x_ref, w_ref, b_ref, h_ref):
    h_ref[...] = (
        jnp.dot(x_ref[...], w_ref[...], preferred_element_type=jnp.float32)
        + b_ref[...]
    )


def _in_proj(x_p, w_p, b):
    grid = (N // ROWS_BLK,)
    return pl.pallas_call(
        _in_proj_body,
        grid=grid,
        in_specs=[
            pl.BlockSpec((ROWS_BLK, 8), lambda i: (i, 0)),
            pl.BlockSpec((8, HID), lambda i: (0, 0)),
            pl.BlockSpec((1, HID), lambda i: (0, 0)),
        ],
        out_specs=pl.BlockSpec((ROWS_BLK, HID), lambda i: (i, 0)),
        out_shape=jax.ShapeDtypeStruct((N, HID), jnp.float32),
    )(x_p, w_p, b)


DEG_BLK = 2048


def _deg_body(p_ref, dis_ref, d2_ref):
    deg = p_ref[0, :, 0:1] + p_ref[1, :, 0:1] + 1.0
    dis = jax.lax.rsqrt(deg)
    dis_ref[...] = dis
    d2_ref[...] = dis * dis


def _deg_reduce(p):
    grid = (NP // DEG_BLK,)
    return pl.pallas_call(
        _deg_body,
        grid=grid,
        in_specs=[
            pl.BlockSpec((2, DEG_BLK, HID), lambda i: (0, i, 0)),
        ],
        out_specs=[
            pl.BlockSpec((DEG_BLK, 1), lambda i: (i, 0)),
            pl.BlockSpec((DEG_BLK, 1), lambda i: (i, 0)),
        ],
        out_shape=[
            jax.ShapeDtypeStruct((NP, 1), jnp.float32),
            jax.ShapeDtypeStruct((NP, 1), jnp.float32),
        ],
    )(p)


def _matmul_body(h_ref, w_ref, o_ref):
    o_ref[...] = jnp.dot(h_ref[...], w_ref[...], preferred_element_type=jnp.float32)


def _matmul(h, w):
    grid = (N // ROWS_BLK,)
    return pl.pallas_call(
        _matmul_body,
        grid=grid,
        in_specs=[
            pl.BlockSpec((ROWS_BLK, HID), lambda i: (i, 0)),
            pl.BlockSpec((HID, HID), lambda i: (0, 0)),
        ],
        out_specs=pl.BlockSpec((ROWS_BLK, HID), lambda i: (i, 0)),
        out_shape=jax.ShapeDtypeStruct((N, HID), jnp.float32),
    )(h, w)


def _post_body(h_ref, agg_ref, m_ref, d2_ref, bc_ref, g_ref, be_ref, o_ref):
    agg = agg_ref[0] + agg_ref[1] + d2_ref[...] * m_ref[...] + bc_ref[...]
    mu = jnp.mean(agg, axis=-1, keepdims=True)
    var = jnp.mean((agg - mu) * (agg - mu), axis=-1, keepdims=True)
    ln = (agg - mu) * jax.lax.rsqrt(var + 1e-5) * g_ref[...] + be_ref[...]
    o_ref[...] = h_ref[...] + _gelu(ln)


def _post(h, agg2, m, d2, bc, g, be):
    grid = (N // ROWS_BLK,)
    return pl.pallas_call(
        _post_body,
        grid=grid,
        in_specs=[
            pl.BlockSpec((ROWS_BLK, HID), lambda i: (i, 0)),
            pl.BlockSpec((2, ROWS_BLK, HID), lambda i: (0, i, 0)),
            pl.BlockSpec((ROWS_BLK, HID), lambda i: (i, 0)),
            pl.BlockSpec((ROWS_BLK, 1), lambda i: (i, 0)),
            pl.BlockSpec((1, HID), lambda i: (0, 0)),
            pl.BlockSpec((1, HID), lambda i: (0, 0)),
            pl.BlockSpec((1, HID), lambda i: (0, 0)),
        ],
        out_specs=pl.BlockSpec((ROWS_BLK, HID), lambda i: (i, 0)),
        out_shape=jax.ShapeDtypeStruct((N, HID), jnp.float32),
    )(h, agg2, m, d2, bc, g, be)


def _head_body(z_ref, w1_ref, b1_ref, w2_ref, b2_ref, w3_ref, b3_ref, o_ref):
    z = _gelu(
        jnp.dot(z_ref[...], w1_ref[...], preferred_element_type=jnp.float32)
        + b1_ref[...]
    )
    z = _gelu(
        jnp.dot(z, w2_ref[...], preferred_element_type=jnp.float32) + b2_ref[...]
    )
    o_ref[...] = (
        jnp.dot(z, w3_ref[...], preferred_element_type=jnp.float32) + b3_ref[...]
    )


def _head(z, w1, b1, w2, b2, w3_p, b3_p):
    bs = N // NPG
    grid = (bs // HEAD_BLK,)
    return pl.pallas_call(
        _head_body,
        grid=grid,
        in_specs=[
            pl.BlockSpec((HEAD_BLK, HID * NPG), lambda i: (i, 0)),
            pl.BlockSpec((HID * NPG, HID * 2), lambda i: (0, 0)),
            pl.BlockSpec((1, HID * 2), lambda i: (0, 0)),
            pl.BlockSpec((HID * 2, HID), lambda i: (0, 0)),
            pl.BlockSpec((1, HID), lambda i: (0, 0)),
            pl.BlockSpec((HID, 128), lambda i: (0, 0)),
            pl.BlockSpec((1, 128), lambda i: (0, 0)),
        ],
        out_specs=pl.BlockSpec((HEAD_BLK, 128), lambda i: (i, 0)),
        out_shape=jax.ShapeDtypeStruct((bs, 128), jnp.float32),
    )(z, w1, b1, w2, b2, w3_p, b3_p)


def kernel(x, edge_index, batch, W_in, b_in, W_c0, b_c0, g0, be0, W_c1, b_c1, g1,
           be1, W_c2, b_c2, g2, be2, fc_W1, fc_b1, fc_W2, fc_b2, fc_W3, fc_b3):
    src = edge_index[0]
    dst = edge_index[1]
    srcp = jnp.pad(src, (0, EP - E))
    dstp = jnp.pad(dst, (0, EP - E))

    ones_m = jnp.ones((N, HID), jnp.float32)
    ones_norm = (jnp.arange(EP, dtype=jnp.int32) < E).astype(jnp.float32)
    degp = _sc_aggregate(ones_m, srcp, dstp, ones_norm)

    x_p = jnp.pad(x, ((0, 0), (0, 5)))
    w_in_p = jnp.pad(W_in, ((0, 5), (0, 0)))
    h = _in_proj(x_p, w_in_p, b_in.reshape(1, HID))
    dis2d, d22d = _deg_reduce(degp)

    normp = _sc_norm(dis2d.reshape(NP), srcp, dstp)
    d2 = d22d[:N]

    for Wc, bc, g, be in (
        (W_c0, b_c0, g0, be0),
        (W_c1, b_c1, g1, be1),
        (W_c2, b_c2, g2, be2),
    ):
        m = _matmul(h, Wc)
        agg2 = _sc_aggregate(m, srcp, dstp, normp)
        h = _post(h, agg2, m, d2, bc.reshape(1, HID), g.reshape(1, HID),
                  be.reshape(1, HID))

    z = h.reshape(N // NPG, HID * NPG)
    w3_p = jnp.pad(fc_W3, ((0, 0), (0, 123)))
    b3_p = jnp.pad(fc_b3, ((0, 123))).reshape(1, 128)
    out = _head(z, fc_W1, fc_b1.reshape(1, HID * 2), fc_W2,
                fc_b2.reshape(1, HID), w3_p, b3_p)
    return out[:, :5]


# speedup vs baseline: 1.1681x; 1.1681x over previous
import jax
import jax.numpy as jnp
from jax import lax
from jax.experimental import pallas as pl
from jax.experimental.pallas import tpu as pltpu
from jax.experimental.pallas import tpu_sc as plsc

N = 100000
E = 400000
HID = 64
NPG = 5
ROWS_BLK = 2000
HEAD_BLK = 2000

EP = 400384
EPT = EP // 32
EPTB = EPT + 32
NG = EPT // 16
NCHUNK = 8
NP = 102400
CHUNK = NP // NCHUNK
STRIPE = CHUNK // 16
ZR = 80
EB = 256
NB = EPTB // EB


def _wid(c, s):
    return c * 16 + s


def _sc_norm_body(dis_hbm, src_hbm, dst_hbm, out_hbm,
                  src_v, dst_v, nrm_v, bs_v, bd_v, ga, gb, sem, sem2):
    c = lax.axis_index("c")
    s = lax.axis_index("s")
    wid = _wid(c, s)
    base = wid * EPT
    pltpu.sync_copy(src_hbm.at[pl.ds(base, EPT)], src_v.at[pl.ds(0, EPT)])
    pltpu.sync_copy(dst_hbm.at[pl.ds(base, EPT)], dst_v.at[pl.ds(0, EPT)])

    z16i = jnp.zeros((16,), jnp.int32)
    for g in range(EPT // 16, EPTB // 16):
        src_v[pl.ds(g * 16, 16)] = z16i
        dst_v[pl.ds(g * 16, 16)] = z16i

    def batch(bi, carry):
        for j in range(EB // 16):
            bs_v[pl.ds(j * 16, 16)] = src_v[pl.ds(bi * EB + j * 16, 16)]
            bd_v[pl.ds(j * 16, 16)] = dst_v[pl.ds(bi * EB + j * 16, 16)]
        cp1 = pltpu.async_copy(dis_hbm.at[bs_v], ga, sem)
        cp2 = pltpu.async_copy(dis_hbm.at[bd_v], gb, sem2)
        cp1.wait()
        cp2.wait()
        for j in range(EB // 16):
            p = (base + bi * EB + j * 16
                 + lax.broadcasted_iota(jnp.int32, (16,), 0))
            v = ga[pl.ds(j * 16, 16)] * gb[pl.ds(j * 16, 16)]
            nrm_v[pl.ds(bi * EB + j * 16, 16)] = jnp.where(p < E, v, 0.0)
        return carry
    lax.fori_loop(0, EPTB // EB, batch, 0)

    pltpu.sync_copy(nrm_v.at[pl.ds(0, EPT)], out_hbm.at[pl.ds(base, EPT)])


def _sc_norm(dis, srcp, dstp):
    return pl.kernel(
        _sc_norm_body,
        out_type=jax.ShapeDtypeStruct((EP,), jnp.float32),
        mesh=plsc.VectorSubcoreMesh(core_axis_name="c", subcore_axis_name="s"),
        compiler_params=pltpu.CompilerParams(use_tc_tiling_on_sc=False),
        scratch_types=[
            pltpu.VMEM((EPTB,), jnp.int32),
            pltpu.VMEM((EPTB,), jnp.int32),
            pltpu.VMEM((EPTB,), jnp.float32),
            pltpu.VMEM((EB,), jnp.int32),
            pltpu.VMEM((EB,), jnp.int32),
            pltpu.VMEM((EB,), jnp.float32),
            pltpu.VMEM((EB,), jnp.float32),
            pltpu.SemaphoreType.DMA,
            pltpu.SemaphoreType.DMA,
        ],
    )(dis, srcp, dstp)


def _sc_agg_body(m_hbm, src_hbm, dst_hbm, norm_hbm, out_hbm,
                 src_v, dst_v, norm_v, bsrc, bsrc2, bdst, bnrm, rows, rows2,
                 zrow, acc, sem, sem2):
    c = lax.axis_index("c")
    s = lax.axis_index("s")
    base_e = _wid(c, s) * EPT
    pltpu.sync_copy(src_hbm.at[pl.ds(base_e, EPT)], src_v.at[pl.ds(0, EPT)])
    pltpu.sync_copy(dst_hbm.at[pl.ds(base_e, EPT)], dst_v.at[pl.ds(0, EPT)])
    pltpu.sync_copy(norm_hbm.at[pl.ds(base_e, EPT)], norm_v.at[pl.ds(0, EPT)])

    z16i = jnp.zeros((16,), jnp.int32)
    z16f = jnp.zeros((16,), jnp.float32)
    for g in range(EPT // 16, EPTB // 16):
        src_v[pl.ds(g * 16, 16)] = z16i
        dst_v[pl.ds(g * 16, 16)] = z16i
        norm_v[pl.ds(g * 16, 16)] = z16f

    def init_zrow(r, carry):
        for j in range(HID // 16):
            zrow[r, pl.ds(j * 16, 16)] = z16f
        return carry
    lax.fori_loop(0, ZR, init_zrow, 0)

    def zero_stripe():
        for k in range(STRIPE // ZR):
            pltpu.sync_copy(zrow, acc.at[pl.ds(s * STRIPE + k * ZR, ZR), :])

    zero_stripe()
    plsc.subcore_barrier()

    for cidx in range(NCHUNK):
        lo = cidx * CHUNK

        def stage_idx(bi, buf):
            for j in range(EB // 16):
                buf[pl.ds(j * 16, 16)] = src_v[pl.ds(bi * EB + j * 16, 16)]

        def gissue(buf, rbuf, sem_):
            pltpu.async_copy(m_hbm.at[buf], rbuf, sem_)

        def gwait(buf, rbuf, sem_):
            pltpu.make_async_copy(m_hbm.at[buf], rbuf, sem_).wait()

        def process(bi, rbuf):
            for j in range(EB // 16):
                d16 = dst_v[pl.ds(bi * EB + j * 16, 16)]
                n16 = norm_v[pl.ds(bi * EB + j * 16, 16)]
                msk = (d16 >= lo) & (d16 < lo + CHUNK)
                bdst[pl.ds(j * 16, 16)] = jnp.clip(d16 - lo, 0, CHUNK - 1)
                bnrm[pl.ds(j * 16, 16)] = jnp.where(msk, n16, 0.0)

            def scale(gg, carry2):
                nv = bnrm[pl.ds(gg * 16, 16)]
                for k in range(16):
                    r = gg * 16 + k
                    nrm = nv[k]
                    for j in range(HID // 16):
                        rbuf[r, pl.ds(j * 16, 16)] = (
                            rbuf[r, pl.ds(j * 16, 16)] * nrm
                        )
                return carry2
            lax.fori_loop(0, EB // 16, scale, 0)
            pltpu.sync_copy(rbuf, acc.at[bdst], add=True)

        stage_idx(0, bsrc)
        gissue(bsrc, rows, sem)

        def batch(bi, carry):
            nxt = bi + 1

            @pl.when(jnp.logical_and(nxt < NB, nxt % 2 == 0))
            def _():
                stage_idx(nxt, bsrc)
                gissue(bsrc, rows, sem)

            @pl.when(jnp.logical_and(nxt < NB, nxt % 2 == 1))
            def _():
                stage_idx(nxt, bsrc2)
                gissue(bsrc2, rows2, sem2)

            @pl.when(bi % 2 == 0)
            def _():
                gwait(bsrc, rows, sem)
                process(bi, rows)

            @pl.when(bi % 2 == 1)
            def _():
                gwait(bsrc2, rows2, sem2)
                process(bi, rows2)
            return carry
        lax.fori_loop(0, NB, batch, 0)

        plsc.subcore_barrier()
        for k in range(STRIPE // ZR):
            pltpu.sync_copy(acc.at[pl.ds(s * STRIPE + k * ZR, ZR), :],
                            out_hbm.at[c, pl.ds(lo + s * STRIPE + k * ZR, ZR), :])
        zero_stripe()
        plsc.subcore_barrier()


def _sc_aggregate(m, srcp, dstp, normp):
    return pl.kernel(
        _sc_agg_body,
        out_type=jax.ShapeDtypeStruct((2, NP, HID), jnp.float32),
        mesh=plsc.VectorSubcoreMesh(core_axis_name="c", subcore_axis_name="s"),
        compiler_params=pltpu.CompilerParams(use_tc_tiling_on_sc=False),
        scratch_types=[
            pltpu.VMEM((EPTB,), jnp.int32),
            pltpu.VMEM((EPTB,), jnp.int32),
            pltpu.VMEM((EPTB,), jnp.float32),
            pltpu.VMEM((EB,), jnp.int32),
            pltpu.VMEM((EB,), jnp.int32),
            pltpu.VMEM((EB,), jnp.int32),
            pltpu.VMEM((EB,), jnp.float32),
            pltpu.VMEM((EB, HID), jnp.float32),
            pltpu.VMEM((EB, HID), jnp.float32),
            pltpu.VMEM((ZR, HID), jnp.float32),
            pltpu.VMEM_SHARED((CHUNK, HID), jnp.float32),
            pltpu.SemaphoreType.DMA,
            pltpu.SemaphoreType.DMA,
        ],
    )(m, srcp, dstp, normp)


def _gelu(t):
    return 0.5 * t * (1.0 + jax.lax.erf(t * 0.7071067811865475))


def _in_proj_body(x_ref, w_ref, b_ref, h_ref):
    h_ref[...] = (
        jnp.dot(x_ref[...], w_ref[...], preferred_element_type=jnp.float32)
        + b_ref[...]
    )


def _in_proj(x_p, w_p, b):
    grid = (N // ROWS_BLK,)
    return pl.pallas_call(
        _in_proj_body,
        grid=grid,
        in_specs=[
            pl.BlockSpec((ROWS_BLK, 8), lambda i: (i, 0)),
            pl.BlockSpec((8, HID), lambda i: (0, 0)),
            pl.BlockSpec((1, HID), lambda i: (0, 0)),
        ],
        out_specs=pl.BlockSpec((ROWS_BLK, HID), lambda i: (i, 0)),
        out_shape=jax.ShapeDtypeStruct((N, HID), jnp.float32),
    )(x_p, w_p, b)


DEG_BLK = 2048


def _deg_body(p_ref, dis_ref, d2_ref):
    deg = p_ref[0, :, 0:1] + p_ref[1, :, 0:1] + 1.0
    dis = jax.lax.rsqrt(deg)
    dis_ref[...] = dis
    d2_ref[...] = dis * dis


def _deg_reduce(p):
    grid = (NP // DEG_BLK,)
    return pl.pallas_call(
        _deg_body,
        grid=grid,
        in_specs=[
            pl.BlockSpec((2, DEG_BLK, HID), lambda i: (0, i, 0)),
        ],
        out_specs=[
            pl.BlockSpec((DEG_BLK, 1), lambda i: (i, 0)),
            pl.BlockSpec((DEG_BLK, 1), lambda i: (i, 0)),
        ],
        out_shape=[
            jax.ShapeDtypeStruct((NP, 1), jnp.float32),
            jax.ShapeDtypeStruct((NP, 1), jnp.float32),
        ],
    )(p)


def _matmul_body(h_ref, w_ref, o_ref):
    o_ref[...] = jnp.dot(h_ref[...], w_ref[...], preferred_element_type=jnp.float32)


def _matmul(h, w):
    grid = (N // ROWS_BLK,)
    return pl.pallas_call(
        _matmul_body,
        grid=grid,
        in_specs=[
            pl.BlockSpec((ROWS_BLK, HID), lambda i: (i, 0)),
            pl.BlockSpec((HID, HID), lambda i: (0, 0)),
        ],
        out_specs=pl.BlockSpec((ROWS_BLK, HID), lambda i: (i, 0)),
        out_shape=jax.ShapeDtypeStruct((N, HID), jnp.float32),
    )(h, w)


def _post_body(h_ref, agg_ref, m_ref, d2_ref, bc_ref, g_ref, be_ref, o_ref):
    agg = agg_ref[0] + agg_ref[1] + d2_ref[...] * m_ref[...] + bc_ref[...]
    mu = jnp.mean(agg, axis=-1, keepdims=True)
    var = jnp.mean((agg - mu) * (agg - mu), axis=-1, keepdims=True)
    ln = (agg - mu) * jax.lax.rsqrt(var + 1e-5) * g_ref[...] + be_ref[...]
    o_ref[...] = h_ref[...] + _gelu(ln)


def _post(h, agg2, m, d2, bc, g, be):
    grid = (N // ROWS_BLK,)
    return pl.pallas_call(
        _post_body,
        grid=grid,
        in_specs=[
            pl.BlockSpec((ROWS_BLK, HID), lambda i: (i, 0)),
            pl.BlockSpec((2, ROWS_BLK, HID), lambda i: (0, i, 0)),
            pl.BlockSpec((ROWS_BLK, HID), lambda i: (i, 0)),
            pl.BlockSpec((ROWS_BLK, 1), lambda i: (i, 0)),
            pl.BlockSpec((1, HID), lambda i: (0, 0)),
            pl.BlockSpec((1, HID), lambda i: (0, 0)),
            pl.BlockSpec((1, HID), lambda i: (0, 0)),
        ],
        out_specs=pl.BlockSpec((ROWS_BLK, HID), lambda i: (i, 0)),
        out_shape=jax.ShapeDtypeStruct((N, HID), jnp.float32),
    )(h, agg2, m, d2, bc, g, be)


def _head_body(z_ref, w1_ref, b1_ref, w2_ref, b2_ref, w3_ref, b3_ref, o_ref):
    z = _gelu(
        jnp.dot(z_ref[...], w1_ref[...], preferred_element_type=jnp.float32)
        + b1_ref[...]
    )
    z = _gelu(
        jnp.dot(z, w2_ref[...], preferred_element_type=jnp.float32) + b2_ref[...]
    )
    o_ref[...] = (
        jnp.dot(z, w3_ref[...], preferred_element_type=jnp.float32) + b3_ref[...]
    )


def _head(z, w1, b1, w2, b2, w3_p, b3_p):
    bs = N // NPG
    grid = (bs // HEAD_BLK,)
    return pl.pallas_call(
        _head_body,
        grid=grid,
        in_specs=[
            pl.BlockSpec((HEAD_BLK, HID * NPG), lambda i: (i, 0)),
            pl.BlockSpec((HID * NPG, HID * 2), lambda i: (0, 0)),
            pl.BlockSpec((1, HID * 2), lambda i: (0, 0)),
            pl.BlockSpec((HID * 2, HID), lambda i: (0, 0)),
            pl.BlockSpec((1, HID), lambda i: (0, 0)),
            pl.BlockSpec((HID, 128), lambda i: (0, 0)),
            pl.BlockSpec((1, 128), lambda i: (0, 0)),
        ],
        out_specs=pl.BlockSpec((HEAD_BLK, 128), lambda i: (i, 0)),
        out_shape=jax.ShapeDtypeStruct((bs, 128), jnp.float32),
    )(z, w1, b1, w2, b2, w3_p, b3_p)


def kernel(x, edge_index, batch, W_in, b_in, W_c0, b_c0, g0, be0, W_c1, b_c1, g1,
           be1, W_c2, b_c2, g2, be2, fc_W1, fc_b1, fc_W2, fc_b2, fc_W3, fc_b3):
    src = edge_index[0]
    dst = edge_index[1]
    srcp = jnp.pad(src, (0, EP - E))
    dstp = jnp.pad(dst, (0, EP - E))

    ones_m = jnp.ones((N, HID), jnp.float32)
    ones_norm = (jnp.arange(EP, dtype=jnp.int32) < E).astype(jnp.float32)
    degp = _sc_aggregate(ones_m, srcp, dstp, ones_norm)

    x_p = jnp.pad(x, ((0, 0), (0, 5)))
    w_in_p = jnp.pad(W_in, ((0, 5), (0, 0)))
    h = _in_proj(x_p, w_in_p, b_in.reshape(1, HID))
    dis2d, d22d = _deg_reduce(degp)

    normp = _sc_norm(dis2d.reshape(NP), srcp, dstp)
    d2 = d22d[:N]

    for Wc, bc, g, be in (
        (W_c0, b_c0, g0, be0),
        (W_c1, b_c1, g1, be1),
        (W_c2, b_c2, g2, be2),
    ):
        m = _matmul(h, Wc)
        agg2 = _sc_aggregate(m, srcp, dstp, normp)
        h = _post(h, agg2, m, d2, bc.reshape(1, HID), g.reshape(1, HID),
                  be.reshape(1, HID))

    z = h.reshape(N // NPG, HID * NPG)
    w3_p = jnp.pad(fc_W3, ((0, 0), (0, 123)))
    b3_p = jnp.pad(fc_b3, ((0, 123))).reshape(1, 128)
    out = _head(z, fc_W1, fc_b1.reshape(1, HID * 2), fc_W2,
                fc_b2.reshape(1, HID), w3_p, b3_p)
    return out[:, :5]


# speedup vs baseline: 1.2820x; 1.0975x over previous
import jax
import jax.numpy as jnp
from jax import lax
from jax.experimental import pallas as pl
from jax.experimental.pallas import tpu as pltpu
from jax.experimental.pallas import tpu_sc as plsc

N = 100000
E = 400000
HID = 64
NPG = 5
ROWS_BLK = 2000
HEAD_BLK = 2000

EP = 400384
EPT = EP // 32
EPTB = EPT + 32
NG = EPT // 16
NCHUNK = 8
NP = 102400
CHUNK = NP // NCHUNK
STRIPE = CHUNK // 16
ZR = 80
EB = 256
NB = EPTB // EB


def _wid(c, s):
    return c * 16 + s


def _sc_norm_body(dis_hbm, src_hbm, dst_hbm, out_hbm,
                  src_v, dst_v, nrm_v, bs_v, bd_v, ga, gb, sem, sem2):
    c = lax.axis_index("c")
    s = lax.axis_index("s")
    wid = _wid(c, s)
    base = wid * EPT
    pltpu.sync_copy(src_hbm.at[pl.ds(base, EPT)], src_v.at[pl.ds(0, EPT)])
    pltpu.sync_copy(dst_hbm.at[pl.ds(base, EPT)], dst_v.at[pl.ds(0, EPT)])

    z16i = jnp.zeros((16,), jnp.int32)
    for g in range(EPT // 16, EPTB // 16):
        src_v[pl.ds(g * 16, 16)] = z16i
        dst_v[pl.ds(g * 16, 16)] = z16i

    def batch(bi, carry):
        for j in range(EB // 16):
            bs_v[pl.ds(j * 16, 16)] = src_v[pl.ds(bi * EB + j * 16, 16)]
            bd_v[pl.ds(j * 16, 16)] = dst_v[pl.ds(bi * EB + j * 16, 16)]
        cp1 = pltpu.async_copy(dis_hbm.at[bs_v], ga, sem)
        cp2 = pltpu.async_copy(dis_hbm.at[bd_v], gb, sem2)
        cp1.wait()
        cp2.wait()
        for j in range(EB // 16):
            p = (base + bi * EB + j * 16
                 + lax.broadcasted_iota(jnp.int32, (16,), 0))
            v = ga[pl.ds(j * 16, 16)] * gb[pl.ds(j * 16, 16)]
            nrm_v[pl.ds(bi * EB + j * 16, 16)] = jnp.where(p < E, v, 0.0)
        return carry
    lax.fori_loop(0, EPTB // EB, batch, 0)

    pltpu.sync_copy(nrm_v.at[pl.ds(0, EPT)], out_hbm.at[pl.ds(base, EPT)])


def _sc_norm(dis, srcp, dstp):
    return pl.kernel(
        _sc_norm_body,
        out_type=jax.ShapeDtypeStruct((EP,), jnp.float32),
        mesh=plsc.VectorSubcoreMesh(core_axis_name="c", subcore_axis_name="s"),
        compiler_params=pltpu.CompilerParams(use_tc_tiling_on_sc=False),
        scratch_types=[
            pltpu.VMEM((EPTB,), jnp.int32),
            pltpu.VMEM((EPTB,), jnp.int32),
            pltpu.VMEM((EPTB,), jnp.float32),
            pltpu.VMEM((EB,), jnp.int32),
            pltpu.VMEM((EB,), jnp.int32),
            pltpu.VMEM((EB,), jnp.float32),
            pltpu.VMEM((EB,), jnp.float32),
            pltpu.SemaphoreType.DMA,
            pltpu.SemaphoreType.DMA,
        ],
    )(dis, srcp, dstp)


def _make_agg_body(hid):
  def _sc_agg_body(m_hbm, src_hbm, dst_hbm, norm_hbm, out_hbm,
                     src_v, dst_v, norm_v, bsrc, bsrc2, bdst, bnrm, rows, rows2,
                     zrow, acc, sem, sem2):
      c = lax.axis_index("c")
      s = lax.axis_index("s")
      base_e = _wid(c, s) * EPT
      pltpu.sync_copy(src_hbm.at[pl.ds(base_e, EPT)], src_v.at[pl.ds(0, EPT)])
      pltpu.sync_copy(dst_hbm.at[pl.ds(base_e, EPT)], dst_v.at[pl.ds(0, EPT)])
      pltpu.sync_copy(norm_hbm.at[pl.ds(base_e, EPT)], norm_v.at[pl.ds(0, EPT)])

      z16i = jnp.zeros((16,), jnp.int32)
      z16f = jnp.zeros((16,), jnp.float32)
      for g in range(EPT // 16, EPTB // 16):
            src_v[pl.ds(g * 16, 16)] = z16i
            dst_v[pl.ds(g * 16, 16)] = z16i
            norm_v[pl.ds(g * 16, 16)] = z16f

      def init_zrow(r, carry):
            for j in range(hid // 16):
                zrow[r, pl.ds(j * 16, 16)] = z16f
            return carry
      lax.fori_loop(0, ZR, init_zrow, 0)

      def zero_stripe():
            for k in range(STRIPE // ZR):
                pltpu.sync_copy(zrow, acc.at[pl.ds(s * STRIPE + k * ZR, ZR), :])

      zero_stripe()
      plsc.subcore_barrier()

      for cidx in range(NCHUNK):
            lo = cidx * CHUNK

            def stage_idx(bi, buf):
                for j in range(EB // 16):
                    buf[pl.ds(j * 16, 16)] = src_v[pl.ds(bi * EB + j * 16, 16)]

            def gissue(buf, rbuf, sem_):
                pltpu.async_copy(m_hbm.at[buf], rbuf, sem_)

            def gwait(buf, rbuf, sem_):
                pltpu.make_async_copy(m_hbm.at[buf], rbuf, sem_).wait()

            def process(bi, rbuf):
                for j in range(EB // 16):
                    d16 = dst_v[pl.ds(bi * EB + j * 16, 16)]
                    n16 = norm_v[pl.ds(bi * EB + j * 16, 16)]
                    msk = (d16 >= lo) & (d16 < lo + CHUNK)
                    bdst[pl.ds(j * 16, 16)] = jnp.clip(d16 - lo, 0, CHUNK - 1)
                    bnrm[pl.ds(j * 16, 16)] = jnp.where(msk, n16, 0.0)

                def scale(gg, carry2):
                    nv = bnrm[pl.ds(gg * 16, 16)]
                    for k in range(16):
                        r = gg * 16 + k
                        nrm = nv[k]
                        for j in range(hid // 16):
                            rbuf[r, pl.ds(j * 16, 16)] = (
                                rbuf[r, pl.ds(j * 16, 16)] * nrm
                            )
                    return carry2
                lax.fori_loop(0, EB // 16, scale, 0)
                pltpu.sync_copy(rbuf, acc.at[bdst], add=True)

            stage_idx(0, bsrc)
            gissue(bsrc, rows, sem)

            def batch(bi, carry):
                nxt = bi + 1

                @pl.when(jnp.logical_and(nxt < NB, nxt % 2 == 0))
                def _():
                    stage_idx(nxt, bsrc)
                    gissue(bsrc, rows, sem)

                @pl.when(jnp.logical_and(nxt < NB, nxt % 2 == 1))
                def _():
                    stage_idx(nxt, bsrc2)
                    gissue(bsrc2, rows2, sem2)

                @pl.when(bi % 2 == 0)
                def _():
                    gwait(bsrc, rows, sem)
                    process(bi, rows)

                @pl.when(bi % 2 == 1)
                def _():
                    gwait(bsrc2, rows2, sem2)
                    process(bi, rows2)
                return carry
            lax.fori_loop(0, NB, batch, 0)

            plsc.subcore_barrier()
            for k in range(STRIPE // ZR):
                pltpu.sync_copy(acc.at[pl.ds(s * STRIPE + k * ZR, ZR), :],
                                out_hbm.at[c, pl.ds(lo + s * STRIPE + k * ZR, ZR), :])
            zero_stripe()
            plsc.subcore_barrier()


  return _sc_agg_body


def _sc_aggregate(m, srcp, dstp, normp, hid=HID):
    return pl.kernel(
        _make_agg_body(hid),
        out_type=jax.ShapeDtypeStruct((2, NP, hid), jnp.float32),
        mesh=plsc.VectorSubcoreMesh(core_axis_name="c", subcore_axis_name="s"),
        compiler_params=pltpu.CompilerParams(use_tc_tiling_on_sc=False),
        scratch_types=[
            pltpu.VMEM((EPTB,), jnp.int32),
            pltpu.VMEM((EPTB,), jnp.int32),
            pltpu.VMEM((EPTB,), jnp.float32),
            pltpu.VMEM((EB,), jnp.int32),
            pltpu.VMEM((EB,), jnp.int32),
            pltpu.VMEM((EB,), jnp.int32),
            pltpu.VMEM((EB,), jnp.float32),
            pltpu.VMEM((EB, hid), jnp.float32),
            pltpu.VMEM((EB, hid), jnp.float32),
            pltpu.VMEM((ZR, hid), jnp.float32),
            pltpu.VMEM_SHARED((CHUNK, hid), jnp.float32),
            pltpu.SemaphoreType.DMA,
            pltpu.SemaphoreType.DMA,
        ],
    )(m, srcp, dstp, normp)


def _gelu(t):
    return 0.5 * t * (1.0 + jax.lax.erf(t * 0.7071067811865475))


def _in_proj_body(x_ref, w_ref, b_ref, h_ref):
    h_ref[...] = (
        jnp.dot(x_ref[...], w_ref[...], preferred_element_type=jnp.float32)
        + b_ref[...]
    )


def _in_proj(x_p, w_p, b):
    grid = (N // ROWS_BLK,)
    return pl.pallas_call(
        _in_proj_body,
        grid=grid,
        in_specs=[
            pl.BlockSpec((ROWS_BLK, 8), lambda i: (i, 0)),
            pl.BlockSpec((8, HID), lambda i: (0, 0)),
            pl.BlockSpec((1, HID), lambda i: (0, 0)),
        ],
        out_specs=pl.BlockSpec((ROWS_BLK, HID), lambda i: (i, 0)),
        out_shape=jax.ShapeDtypeStruct((N, HID), jnp.float32),
    )(x_p, w_p, b)


DEG_BLK = 2048


def _deg_body(p_ref, dis_ref, d2_ref):
    deg = p_ref[0, :, 0:1] + p_ref[1, :, 0:1] + 1.0
    dis = jax.lax.rsqrt(deg)
    dis_ref[...] = dis
    d2_ref[...] = dis * dis


def _deg_reduce(p):
    grid = (NP // DEG_BLK,)
    return pl.pallas_call(
        _deg_body,
        grid=grid,
        in_specs=[
            pl.BlockSpec((2, DEG_BLK, 16), lambda i: (0, i, 0)),
        ],
        out_specs=[
            pl.BlockSpec((DEG_BLK, 1), lambda i: (i, 0)),
            pl.BlockSpec((DEG_BLK, 1), lambda i: (i, 0)),
        ],
        out_shape=[
            jax.ShapeDtypeStruct((NP, 1), jnp.float32),
            jax.ShapeDtypeStruct((NP, 1), jnp.float32),
        ],
    )(p)


def _matmul_body(h_ref, w_ref, o_ref):
    o_ref[...] = jnp.dot(h_ref[...], w_ref[...], preferred_element_type=jnp.float32)


def _matmul(h, w):
    grid = (N // ROWS_BLK,)
    return pl.pallas_call(
        _matmul_body,
        grid=grid,
        in_specs=[
            pl.BlockSpec((ROWS_BLK, HID), lambda i: (i, 0)),
            pl.BlockSpec((HID, HID), lambda i: (0, 0)),
        ],
        out_specs=pl.BlockSpec((ROWS_BLK, HID), lambda i: (i, 0)),
        out_shape=jax.ShapeDtypeStruct((N, HID), jnp.float32),
    )(h, w)


def _post_body(h_ref, agg_ref, m_ref, d2_ref, bc_ref, g_ref, be_ref, o_ref):
    agg = agg_ref[0] + agg_ref[1] + d2_ref[...] * m_ref[...] + bc_ref[...]
    mu = jnp.mean(agg, axis=-1, keepdims=True)
    var = jnp.mean((agg - mu) * (agg - mu), axis=-1, keepdims=True)
    ln = (agg - mu) * jax.lax.rsqrt(var + 1e-5) * g_ref[...] + be_ref[...]
    o_ref[...] = h_ref[...] + _gelu(ln)


def _post(h, agg2, m, d2, bc, g, be):
    grid = (N // ROWS_BLK,)
    return pl.pallas_call(
        _post_body,
        grid=grid,
        in_specs=[
            pl.BlockSpec((ROWS_BLK, HID), lambda i: (i, 0)),
            pl.BlockSpec((2, ROWS_BLK, HID), lambda i: (0, i, 0)),
            pl.BlockSpec((ROWS_BLK, HID), lambda i: (i, 0)),
            pl.BlockSpec((ROWS_BLK, 1), lambda i: (i, 0)),
            pl.BlockSpec((1, HID), lambda i: (0, 0)),
            pl.BlockSpec((1, HID), lambda i: (0, 0)),
            pl.BlockSpec((1, HID), lambda i: (0, 0)),
        ],
        out_specs=pl.BlockSpec((ROWS_BLK, HID), lambda i: (i, 0)),
        out_shape=jax.ShapeDtypeStruct((N, HID), jnp.float32),
    )(h, agg2, m, d2, bc, g, be)


def _head_body(z_ref, w1_ref, b1_ref, w2_ref, b2_ref, w3_ref, b3_ref, o_ref):
    z = _gelu(
        jnp.dot(z_ref[...], w1_ref[...], preferred_element_type=jnp.float32)
        + b1_ref[...]
    )
    z = _gelu(
        jnp.dot(z, w2_ref[...], preferred_element_type=jnp.float32) + b2_ref[...]
    )
    o_ref[...] = (
        jnp.dot(z, w3_ref[...], preferred_element_type=jnp.float32) + b3_ref[...]
    )


def _head(z, w1, b1, w2, b2, w3_p, b3_p):
    bs = N // NPG
    grid = (bs // HEAD_BLK,)
    return pl.pallas_call(
        _head_body,
        grid=grid,
        in_specs=[
            pl.BlockSpec((HEAD_BLK, HID * NPG), lambda i: (i, 0)),
            pl.BlockSpec((HID * NPG, HID * 2), lambda i: (0, 0)),
            pl.BlockSpec((1, HID * 2), lambda i: (0, 0)),
            pl.BlockSpec((HID * 2, HID), lambda i: (0, 0)),
            pl.BlockSpec((1, HID), lambda i: (0, 0)),
            pl.BlockSpec((HID, 128), lambda i: (0, 0)),
            pl.BlockSpec((1, 128), lambda i: (0, 0)),
        ],
        out_specs=pl.BlockSpec((HEAD_BLK, 128), lambda i: (i, 0)),
        out_shape=jax.ShapeDtypeStruct((bs, 128), jnp.float32),
    )(z, w1, b1, w2, b2, w3_p, b3_p)


def kernel(x, edge_index, batch, W_in, b_in, W_c0, b_c0, g0, be0, W_c1, b_c1, g1,
           be1, W_c2, b_c2, g2, be2, fc_W1, fc_b1, fc_W2, fc_b2, fc_W3, fc_b3):
    src = edge_index[0]
    dst = edge_index[1]
    srcp = jnp.pad(src, (0, EP - E))
    dstp = jnp.pad(dst, (0, EP - E))

    ones_m = jnp.ones((N, 16), jnp.float32)
    ones_norm = (jnp.arange(EP, dtype=jnp.int32) < E).astype(jnp.float32)
    degp = _sc_aggregate(ones_m, srcp, dstp, ones_norm, hid=16)

    x_p = jnp.pad(x, ((0, 0), (0, 5)))
    w_in_p = jnp.pad(W_in, ((0, 5), (0, 0)))
    h = _in_proj(x_p, w_in_p, b_in.reshape(1, HID))
    dis2d, d22d = _deg_reduce(degp)

    normp = _sc_norm(dis2d.reshape(NP), srcp, dstp)
    d2 = d22d[:N]

    for Wc, bc, g, be in (
        (W_c0, b_c0, g0, be0),
        (W_c1, b_c1, g1, be1),
        (W_c2, b_c2, g2, be2),
    ):
        m = _matmul(h, Wc)
        agg2 = _sc_aggregate(m, srcp, dstp, normp)
        h = _post(h, agg2, m, d2, bc.reshape(1, HID), g.reshape(1, HID),
                  be.reshape(1, HID))

    z = h.reshape(N // NPG, HID * NPG)
    w3_p = jnp.pad(fc_W3, ((0, 0), (0, 123)))
    b3_p = jnp.pad(fc_b3, ((0, 123))).reshape(1, 128)
    out = _head(z, fc_W1, fc_b1.reshape(1, HID * 2), fc_W2,
                fc_b2.reshape(1, HID), w3_p, b3_p)
    return out[:, :5]


# speedup vs baseline: 1.2856x; 1.0028x over previous
import jax
import jax.numpy as jnp
from jax import lax
from jax.experimental import pallas as pl
from jax.experimental.pallas import tpu as pltpu
from jax.experimental.pallas import tpu_sc as plsc

N = 100000
E = 400000
HID = 64
NPG = 5
ROWS_BLK = 2000
HEAD_BLK = 2000

EP = 400384
EPT = EP // 32
EPTB = EPT + 32
NG = EPT // 16
NCHUNK = 8
NP = 102400
CHUNK = NP // NCHUNK
STRIPE = CHUNK // 16
ZR = 80
EB = 256
NB = EPTB // EB


def _wid(c, s):
    return c * 16 + s


def _sc_norm_body(dis_hbm, src_hbm, dst_hbm, out_hbm,
                  src_v, dst_v, nrm_v, bs_v, bd_v, ga, gb, sem, sem2):
    c = lax.axis_index("c")
    s = lax.axis_index("s")
    wid = _wid(c, s)
    base = wid * EPT
    pltpu.sync_copy(src_hbm.at[pl.ds(base, EPT)], src_v.at[pl.ds(0, EPT)])
    pltpu.sync_copy(dst_hbm.at[pl.ds(base, EPT)], dst_v.at[pl.ds(0, EPT)])

    z16i = jnp.zeros((16,), jnp.int32)
    for g in range(EPT // 16, EPTB // 16):
        src_v[pl.ds(g * 16, 16)] = z16i
        dst_v[pl.ds(g * 16, 16)] = z16i

    def batch(bi, carry):
        for j in range(EB // 16):
            bs_v[pl.ds(j * 16, 16)] = src_v[pl.ds(bi * EB + j * 16, 16)]
            bd_v[pl.ds(j * 16, 16)] = dst_v[pl.ds(bi * EB + j * 16, 16)]
        cp1 = pltpu.async_copy(dis_hbm.at[bs_v], ga, sem)
        cp2 = pltpu.async_copy(dis_hbm.at[bd_v], gb, sem2)
        cp1.wait()
        cp2.wait()
        for j in range(EB // 16):
            p = (base + bi * EB + j * 16
                 + lax.broadcasted_iota(jnp.int32, (16,), 0))
            v = ga[pl.ds(j * 16, 16)] * gb[pl.ds(j * 16, 16)]
            nrm_v[pl.ds(bi * EB + j * 16, 16)] = jnp.where(p < E, v, 0.0)
        return carry
    lax.fori_loop(0, EPTB // EB, batch, 0)

    pltpu.sync_copy(nrm_v.at[pl.ds(0, EPT)], out_hbm.at[pl.ds(base, EPT)])


def _sc_norm(dis, srcp, dstp):
    return pl.kernel(
        _sc_norm_body,
        out_type=jax.ShapeDtypeStruct((EP,), jnp.float32),
        mesh=plsc.VectorSubcoreMesh(core_axis_name="c", subcore_axis_name="s"),
        compiler_params=pltpu.CompilerParams(use_tc_tiling_on_sc=False),
        scratch_types=[
            pltpu.VMEM((EPTB,), jnp.int32),
            pltpu.VMEM((EPTB,), jnp.int32),
            pltpu.VMEM((EPTB,), jnp.float32),
            pltpu.VMEM((EB,), jnp.int32),
            pltpu.VMEM((EB,), jnp.int32),
            pltpu.VMEM((EB,), jnp.float32),
            pltpu.VMEM((EB,), jnp.float32),
            pltpu.SemaphoreType.DMA,
            pltpu.SemaphoreType.DMA,
        ],
    )(dis, srcp, dstp)


def _make_agg_body(hid):
  def _sc_agg_body(m_hbm, src_hbm, dst_hbm, norm_hbm, out_hbm,
                     src_v, dst_v, norm_v, bsrc, bsrc2, bdst, bdst2, bnrm,
                     rows, rows2, zrow, acc, sem, sem2, ssem, ssem2):
      c = lax.axis_index("c")
      s = lax.axis_index("s")
      base_e = _wid(c, s) * EPT
      pltpu.sync_copy(src_hbm.at[pl.ds(base_e, EPT)], src_v.at[pl.ds(0, EPT)])
      pltpu.sync_copy(dst_hbm.at[pl.ds(base_e, EPT)], dst_v.at[pl.ds(0, EPT)])
      pltpu.sync_copy(norm_hbm.at[pl.ds(base_e, EPT)], norm_v.at[pl.ds(0, EPT)])

      z16i = jnp.zeros((16,), jnp.int32)
      z16f = jnp.zeros((16,), jnp.float32)
      for g in range(EPT // 16, EPTB // 16):
            src_v[pl.ds(g * 16, 16)] = z16i
            dst_v[pl.ds(g * 16, 16)] = z16i
            norm_v[pl.ds(g * 16, 16)] = z16f

      def init_zrow(r, carry):
            for j in range(hid // 16):
                zrow[r, pl.ds(j * 16, 16)] = z16f
            return carry
      lax.fori_loop(0, ZR, init_zrow, 0)

      def zero_stripe():
            for k in range(STRIPE // ZR):
                pltpu.sync_copy(zrow, acc.at[pl.ds(s * STRIPE + k * ZR, ZR), :])

      zero_stripe()
      plsc.subcore_barrier()

      for cidx in range(NCHUNK):
            lo = cidx * CHUNK

            def stage_idx(bi, buf):
                for j in range(EB // 16):
                    buf[pl.ds(j * 16, 16)] = src_v[pl.ds(bi * EB + j * 16, 16)]

            def gissue(buf, rbuf, sem_):
                pltpu.async_copy(m_hbm.at[buf], rbuf, sem_)

            def gwait(buf, rbuf, sem_):
                pltpu.make_async_copy(m_hbm.at[buf], rbuf, sem_).wait()

            def process(bi, rbuf, bdbuf, ssem_):
                for j in range(EB // 16):
                    d16 = dst_v[pl.ds(bi * EB + j * 16, 16)]
                    n16 = norm_v[pl.ds(bi * EB + j * 16, 16)]
                    msk = (d16 >= lo) & (d16 < lo + CHUNK)
                    bdbuf[pl.ds(j * 16, 16)] = jnp.clip(d16 - lo, 0, CHUNK - 1)
                    bnrm[pl.ds(j * 16, 16)] = jnp.where(msk, n16, 0.0)

                def scale(gg, carry2):
                    nv = bnrm[pl.ds(gg * 16, 16)]
                    for k in range(16):
                        r = gg * 16 + k
                        nrm = nv[k]
                        for j in range(hid // 16):
                            rbuf[r, pl.ds(j * 16, 16)] = (
                                rbuf[r, pl.ds(j * 16, 16)] * nrm
                            )
                    return carry2
                lax.fori_loop(0, EB // 16, scale, 0)
                pltpu.async_copy(rbuf, acc.at[bdbuf], ssem_, add=True)

            def swait(rbuf, bdbuf, ssem_):
                pltpu.make_async_copy(rbuf, acc.at[bdbuf], ssem_).wait()

            stage_idx(0, bsrc)
            gissue(bsrc, rows, sem)

            def batch(bi, carry):
                nxt = bi + 1

                @pl.when(jnp.logical_and(nxt < NB, nxt % 2 == 0))
                def _():
                    @pl.when(nxt >= 2)
                    def _():
                        swait(rows, bdst, ssem)
                    stage_idx(nxt, bsrc)
                    gissue(bsrc, rows, sem)

                @pl.when(jnp.logical_and(nxt < NB, nxt % 2 == 1))
                def _():
                    @pl.when(nxt >= 2)
                    def _():
                        swait(rows2, bdst2, ssem2)
                    stage_idx(nxt, bsrc2)
                    gissue(bsrc2, rows2, sem2)

                @pl.when(bi % 2 == 0)
                def _():
                    gwait(bsrc, rows, sem)
                    process(bi, rows, bdst, ssem)

                @pl.when(bi % 2 == 1)
                def _():
                    gwait(bsrc2, rows2, sem2)
                    process(bi, rows2, bdst2, ssem2)
                return carry
            lax.fori_loop(0, NB, batch, 0)
            swait(rows2, bdst2, ssem2)
            swait(rows, bdst, ssem)

            plsc.subcore_barrier()
            for k in range(STRIPE // ZR):
                pltpu.sync_copy(acc.at[pl.ds(s * STRIPE + k * ZR, ZR), :],
                                out_hbm.at[c, pl.ds(lo + s * STRIPE + k * ZR, ZR), :])
            zero_stripe()
            plsc.subcore_barrier()


  return _sc_agg_body


def _sc_aggregate(m, srcp, dstp, normp, hid=HID):
    return pl.kernel(
        _make_agg_body(hid),
        out_type=jax.ShapeDtypeStruct((2, NP, hid), jnp.float32),
        mesh=plsc.VectorSubcoreMesh(core_axis_name="c", subcore_axis_name="s"),
        compiler_params=pltpu.CompilerParams(use_tc_tiling_on_sc=False),
        scratch_types=[
            pltpu.VMEM((EPTB,), jnp.int32),
            pltpu.VMEM((EPTB,), jnp.int32),
            pltpu.VMEM((EPTB,), jnp.float32),
            pltpu.VMEM((EB,), jnp.int32),
            pltpu.VMEM((EB,), jnp.int32),
            pltpu.VMEM((EB,), jnp.int32),
            pltpu.VMEM((EB,), jnp.int32),
            pltpu.VMEM((EB,), jnp.float32),
            pltpu.VMEM((EB, hid), jnp.float32),
            pltpu.VMEM((EB, hid), jnp.float32),
            pltpu.VMEM((ZR, hid), jnp.float32),
            pltpu.VMEM_SHARED((CHUNK, hid), jnp.float32),
            pltpu.SemaphoreType.DMA,
            pltpu.SemaphoreType.DMA,
            pltpu.SemaphoreType.DMA,
            pltpu.SemaphoreType.DMA,
        ],
    )(m, srcp, dstp, normp)


def _gelu(t):
    return 0.5 * t * (1.0 + jax.lax.erf(t * 0.7071067811865475))


def _in_proj_body(x_ref, w_ref, b_ref, h_ref):
    h_ref[...] = (
        jnp.dot(x_ref[...], w_ref[...], preferred_element_type=jnp.float32)
        + b_ref[...]
    )


def _in_proj(x_p, w_p, b):
    grid = (N // ROWS_BLK,)
    return pl.pallas_call(
        _in_proj_body,
        grid=grid,
        in_specs=[
            pl.BlockSpec((ROWS_BLK, 8), lambda i: (i, 0)),
            pl.BlockSpec((8, HID), lambda i: (0, 0)),
            pl.BlockSpec((1, HID), lambda i: (0, 0)),
        ],
        out_specs=pl.BlockSpec((ROWS_BLK, HID), lambda i: (i, 0)),
        out_shape=jax.ShapeDtypeStruct((N, HID), jnp.float32),
    )(x_p, w_p, b)


DEG_BLK = 2048


def _deg_body(p_ref, dis_ref, d2_ref):
    deg = p_ref[0, :, 0:1] + p_ref[1, :, 0:1] + 1.0
    dis = jax.lax.rsqrt(deg)
    dis_ref[...] = dis
    d2_ref[...] = dis * dis


def _deg_reduce(p):
    grid = (NP // DEG_BLK,)
    return pl.pallas_call(
        _deg_body,
        grid=grid,
        in_specs=[
            pl.BlockSpec((2, DEG_BLK, 16), lambda i: (0, i, 0)),
        ],
        out_specs=[
            pl.BlockSpec((DEG_BLK, 1), lambda i: (i, 0)),
            pl.BlockSpec((DEG_BLK, 1), lambda i: (i, 0)),
        ],
        out_shape=[
            jax.ShapeDtypeStruct((NP, 1), jnp.float32),
            jax.ShapeDtypeStruct((NP, 1), jnp.float32),
        ],
    )(p)


def _matmul_body(h_ref, w_ref, o_ref):
    o_ref[...] = jnp.dot(h_ref[...], w_ref[...], preferred_element_type=jnp.float32)


def _matmul(h, w):
    grid = (N // ROWS_BLK,)
    return pl.pallas_call(
        _matmul_body,
        grid=grid,
        in_specs=[
            pl.BlockSpec((ROWS_BLK, HID), lambda i: (i, 0)),
            pl.BlockSpec((HID, HID), lambda i: (0, 0)),
        ],
        out_specs=pl.BlockSpec((ROWS_BLK, HID), lambda i: (i, 0)),
        out_shape=jax.ShapeDtypeStruct((N, HID), jnp.float32),
    )(h, w)


def _post_body(h_ref, agg_ref, m_ref, d2_ref, bc_ref, g_ref, be_ref, o_ref):
    agg = agg_ref[0] + agg_ref[1] + d2_ref[...] * m_ref[...] + bc_ref[...]
    mu = jnp.mean(agg, axis=-1, keepdims=True)
    var = jnp.mean((agg - mu) * (agg - mu), axis=-1, keepdims=True)
    ln = (agg - mu) * jax.lax.rsqrt(var + 1e-5) * g_ref[...] + be_ref[...]
    o_ref[...] = h_ref[...] + _gelu(ln)


def _post(h, agg2, m, d2, bc, g, be):
    grid = (N // ROWS_BLK,)
    return pl.pallas_call(
        _post_body,
        grid=grid,
        in_specs=[
            pl.BlockSpec((ROWS_BLK, HID), lambda i: (i, 0)),
            pl.BlockSpec((2, ROWS_BLK, HID), lambda i: (0, i, 0)),
            pl.BlockSpec((ROWS_BLK, HID), lambda i: (i, 0)),
            pl.BlockSpec((ROWS_BLK, 1), lambda i: (i, 0)),
            pl.BlockSpec((1, HID), lambda i: (0, 0)),
            pl.BlockSpec((1, HID), lambda i: (0, 0)),
            pl.BlockSpec((1, HID), lambda i: (0, 0)),
        ],
        out_specs=pl.BlockSpec((ROWS_BLK, HID), lambda i: (i, 0)),
        out_shape=jax.ShapeDtypeStruct((N, HID), jnp.float32),
    )(h, agg2, m, d2, bc, g, be)


def _head_body(z_ref, w1_ref, b1_ref, w2_ref, b2_ref, w3_ref, b3_ref, o_ref):
    z = _gelu(
        jnp.dot(z_ref[...], w1_ref[...], preferred_element_type=jnp.float32)
        + b1_ref[...]
    )
    z = _gelu(
        jnp.dot(z, w2_ref[...], preferred_element_type=jnp.float32) + b2_ref[...]
    )
    o_ref[...] = (
        jnp.dot(z, w3_ref[...], preferred_element_type=jnp.float32) + b3_ref[...]
    )


def _head(z, w1, b1, w2, b2, w3_p, b3_p):
    bs = N // NPG
    grid = (bs // HEAD_BLK,)
    return pl.pallas_call(
        _head_body,
        grid=grid,
        in_specs=[
            pl.BlockSpec((HEAD_BLK, HID * NPG), lambda i: (i, 0)),
            pl.BlockSpec((HID * NPG, HID * 2), lambda i: (0, 0)),
            pl.BlockSpec((1, HID * 2), lambda i: (0, 0)),
            pl.BlockSpec((HID * 2, HID), lambda i: (0, 0)),
            pl.BlockSpec((1, HID), lambda i: (0, 0)),
            pl.BlockSpec((HID, 128), lambda i: (0, 0)),
            pl.BlockSpec((1, 128), lambda i: (0, 0)),
        ],
        out_specs=pl.BlockSpec((HEAD_BLK, 128), lambda i: (i, 0)),
        out_shape=jax.ShapeDtypeStruct((bs, 128), jnp.float32),
    )(z, w1, b1, w2, b2, w3_p, b3_p)


def kernel(x, edge_index, batch, W_in, b_in, W_c0, b_c0, g0, be0, W_c1, b_c1, g1,
           be1, W_c2, b_c2, g2, be2, fc_W1, fc_b1, fc_W2, fc_b2, fc_W3, fc_b3):
    src = edge_index[0]
    dst = edge_index[1]
    srcp = jnp.pad(src, (0, EP - E))
    dstp = jnp.pad(dst, (0, EP - E))

    ones_m = jnp.ones((N, 16), jnp.float32)
    ones_norm = (jnp.arange(EP, dtype=jnp.int32) < E).astype(jnp.float32)
    degp = _sc_aggregate(ones_m, srcp, dstp, ones_norm, hid=16)

    x_p = jnp.pad(x, ((0, 0), (0, 5)))
    w_in_p = jnp.pad(W_in, ((0, 5), (0, 0)))
    h = _in_proj(x_p, w_in_p, b_in.reshape(1, HID))
    dis2d, d22d = _deg_reduce(degp)

    normp = _sc_norm(dis2d.reshape(NP), srcp, dstp)
    d2 = d22d[:N]

    for Wc, bc, g, be in (
        (W_c0, b_c0, g0, be0),
        (W_c1, b_c1, g1, be1),
        (W_c2, b_c2, g2, be2),
    ):
        m = _matmul(h, Wc)
        agg2 = _sc_aggregate(m, srcp, dstp, normp)
        h = _post(h, agg2, m, d2, bc.reshape(1, HID), g.reshape(1, HID),
                  be.reshape(1, HID))

    z = h.reshape(N // NPG, HID * NPG)
    w3_p = jnp.pad(fc_W3, ((0, 0), (0, 123)))
    b3_p = jnp.pad(fc_b3, ((0, 123))).reshape(1, 128)
    out = _head(z, fc_W1, fc_b1.reshape(1, HID * 2), fc_W2,
                fc_b2.reshape(1, HID), w3_p, b3_p)
    return out[:, :5]
